# baseline (speedup 1.0000x reference)
import functools

import jax
import jax.numpy as jnp
from jax import lax
from jax.experimental import pallas as pl
from jax.experimental.pallas import tpu as pltpu
from jax.experimental.pallas import tpu_sc as plsc

B = 16384
D = 32
N_CORES = 2
N_SUBCORES = 16
NW = N_CORES * N_SUBCORES
BPW = B // NW
LANES = 16
NBUF = 4

_mesh = plsc.VectorSubcoreMesh(core_axis_name="c", subcore_axis_name="s")


@functools.partial(
    pl.kernel,
    mesh=_mesh,
    out_type=jax.ShapeDtypeStruct((B,), jnp.float32),
    compiler_params=pltpu.CompilerParams(
        needs_layout_passes=False, use_tc_tiling_on_sc=True),
    scratch_types=[
        pltpu.VMEM((BPW + LANES,), jnp.int32),
        pltpu.VMEM((BPW + LANES,), jnp.int32),
        pltpu.VMEM((NBUF, D, 128), jnp.float32),
        pltpu.VMEM((NBUF, D, 128), jnp.float32),
        pltpu.VMEM((BPW,), jnp.float32),
        pltpu.SemaphoreType.DMA,
        pltpu.SemaphoreType.DMA,
    ],
)
def _gcmc_sc_kernel(uid_hbm, iid_hbm, utT_hbm, itT_hbm, out_hbm,
                    uidx, iidx, ublk, iblk, out_v, usem, isem):
    wid = lax.axis_index("s") * N_CORES + lax.axis_index("c")
    base = wid * BPW

    pltpu.sync_copy(uid_hbm.at[pl.ds(base, BPW)], uidx.at[pl.ds(0, BPW)])
    pltpu.sync_copy(iid_hbm.at[pl.ds(base, BPW)], iidx.at[pl.ds(0, BPW)])

    lane = jnp.arange(LANES, dtype=jnp.int32)
    c_lo = lane * 128
    c_hi = c_lo + LANES * 128

    def issue(uvi, ivi, slot):
        tc_u = pl.multiple_of((uvi >> 7) * 128, 128)
        tc_i = pl.multiple_of((ivi >> 7) * 128, 128)
        pltpu.make_async_copy(
            utT_hbm.at[:, pl.ds(tc_u, 128)], ublk.at[slot], usem).start()
        pltpu.make_async_copy(
            itT_hbm.at[:, pl.ds(tc_i, 128)], iblk.at[slot], isem).start()

    def wait(slot):
        pltpu.make_async_copy(
            utT_hbm.at[:, pl.ds(0, 128)], ublk.at[slot], usem).wait()
        pltpu.make_async_copy(
            itT_hbm.at[:, pl.ds(0, 128)], iblk.at[slot], isem).wait()

    uvec0 = uidx[pl.ds(0, LANES)]
    ivec0 = iidx[pl.ds(0, LANES)]
    for n in range(NBUF):
        issue(uvec0[n], ivec0[n], n)

    def body(g, carry):
        uvec = uidx[pl.ds(g * LANES, LANES)]
        ivec = iidx[pl.ds(g * LANES, LANES)]
        uvec_n = uidx[pl.ds((g + 1) * LANES, LANES)]
        ivec_n = iidx[pl.ds((g + 1) * LANES, LANES)]
        acc = jnp.zeros((LANES,), jnp.float32)
        for i in range(LANES):
            slot = i % NBUF
            lu = jnp.full((LANES,), uvec[i] & 127, jnp.int32)
            li = jnp.full((LANES,), ivec[i] & 127, jnp.int32)

            wait(slot)
            u0 = plsc.load_gather(ublk.at[slot], [lane, lu])
            u1 = plsc.load_gather(ublk.at[slot], [lane + LANES, lu])
            v0 = plsc.load_gather(iblk.at[slot], [lane, li])
            v1 = plsc.load_gather(iblk.at[slot], [lane + LANES, li])
            s = jnp.sum(u0 * v0 + u1 * v1)
            acc = jnp.where(lane == i, s, acc)

            if i + NBUF < LANES:
                issue(uvec[i + NBUF], ivec[i + NBUF], slot)
            else:
                @pl.when(g < BPW // LANES - 1)
                def _():
                    issue(uvec_n[i + NBUF - LANES],
                          ivec_n[i + NBUF - LANES], slot)

        out_v[pl.ds(g * LANES, LANES)] = acc
        return carry

    lax.fori_loop(0, BPW // LANES, body, 0)

    pltpu.sync_copy(out_v, out_hbm.at[pl.ds(base, BPW)])


def kernel(x, user_embedding, item_embedding):
    uid = x[:, 0]
    iid = x[:, 1]
    return _gcmc_sc_kernel(uid, iid, user_embedding.T, item_embedding.T)


# speedup vs baseline: 1.1314x; 1.1314x over previous
import functools

import jax
import jax.numpy as jnp
from jax import lax
from jax.experimental import pallas as pl
from jax.experimental.pallas import tpu as pltpu
from jax.experimental.pallas import tpu_sc as plsc

B = 16384
D = 32
N_CORES = 2
N_SUBCORES = 16
NW = N_CORES * N_SUBCORES
BPW = B // NW
LANES = 16
NBUF = 8
NBUF_U = 16

_mesh = plsc.VectorSubcoreMesh(core_axis_name="c", subcore_axis_name="s")


@functools.partial(
    pl.kernel,
    mesh=_mesh,
    out_type=jax.ShapeDtypeStruct((B,), jnp.float32),
    compiler_params=pltpu.CompilerParams(
        needs_layout_passes=False, use_tc_tiling_on_sc=True),
    scratch_types=[
        pltpu.VMEM((BPW + LANES,), jnp.int32),
        pltpu.VMEM((BPW + LANES,), jnp.int32),
        pltpu.VMEM((NBUF_U, D, 128), jnp.float32),
        pltpu.VMEM((NBUF, D, 128), jnp.float32),
        pltpu.VMEM((BPW,), jnp.float32),
        pltpu.SemaphoreType.DMA,
        pltpu.SemaphoreType.DMA,
    ],
)
def _gcmc_sc_kernel(uid_hbm, iid_hbm, utT_hbm, itT_hbm, out_hbm,
                    uidx, iidx, ublk, iblk, out_v, usem, isem):
    wid = lax.axis_index("s") * N_CORES + lax.axis_index("c")
    base = wid * BPW

    pltpu.sync_copy(uid_hbm.at[pl.ds(base, BPW)], uidx.at[pl.ds(0, BPW)])
    pltpu.sync_copy(iid_hbm.at[pl.ds(base, BPW)], iidx.at[pl.ds(0, BPW)])

    lane = jnp.arange(LANES, dtype=jnp.int32)
    c_lo = lane * 128
    c_hi = c_lo + LANES * 128

    def issue_u(uvi, slot):
        tc_u = pl.multiple_of((uvi >> 7) * 128, 128)
        pltpu.make_async_copy(
            utT_hbm.at[:, pl.ds(tc_u, 128)], ublk.at[slot], usem).start()

    def issue_i(ivi, slot):
        tc_i = pl.multiple_of((ivi >> 7) * 128, 128)
        pltpu.make_async_copy(
            itT_hbm.at[:, pl.ds(tc_i, 128)], iblk.at[slot], isem).start()

    def wait_u(slot):
        pltpu.make_async_copy(
            utT_hbm.at[:, pl.ds(0, 128)], ublk.at[slot], usem).wait()

    def wait_i(slot):
        pltpu.make_async_copy(
            itT_hbm.at[:, pl.ds(0, 128)], iblk.at[slot], isem).wait()

    uvec0 = uidx[pl.ds(0, LANES)]
    ivec0 = iidx[pl.ds(0, LANES)]
    uvec1 = uidx[pl.ds(LANES, LANES)]
    for n in range(NBUF_U):
        if n < LANES:
            issue_u(uvec0[n], n)
        else:
            issue_u(uvec1[n - LANES], n)
    for n in range(NBUF):
        issue_i(ivec0[n], n)

    def body(g, carry):
        uvec = uidx[pl.ds(g * LANES, LANES)]
        ivec = iidx[pl.ds(g * LANES, LANES)]
        uvec_n = uidx[pl.ds((g + 1) * LANES, LANES)]
        ivec_n = iidx[pl.ds((g + 1) * LANES, LANES)]
        acc = jnp.zeros((LANES,), jnp.float32)
        for i in range(LANES):
            slot_u = i % NBUF_U
            slot_i = i % NBUF
            lu = jnp.full((LANES,), uvec[i] & 127, jnp.int32)
            li = jnp.full((LANES,), ivec[i] & 127, jnp.int32)

            wait_u(slot_u)
            wait_i(slot_i)
            u0 = plsc.load_gather(ublk.at[slot_u], [lane, lu])
            u1 = plsc.load_gather(ublk.at[slot_u], [lane + LANES, lu])
            v0 = plsc.load_gather(iblk.at[slot_i], [lane, li])
            v1 = plsc.load_gather(iblk.at[slot_i], [lane + LANES, li])
            s = jnp.sum(u0 * v0 + u1 * v1)
            acc = jnp.where(lane == i, s, acc)

            @pl.when(g < BPW // LANES - 1)
            def _():
                issue_u(uvec_n[i], slot_u)

            if i + NBUF < LANES:
                issue_i(ivec[i + NBUF], slot_i)
            else:
                @pl.when(g < BPW // LANES - 1)
                def _():
                    issue_i(ivec_n[i + NBUF - LANES], slot_i)

        out_v[pl.ds(g * LANES, LANES)] = acc
        return carry

    lax.fori_loop(0, BPW // LANES, body, 0)

    pltpu.sync_copy(out_v, out_hbm.at[pl.ds(base, BPW)])


def kernel(x, user_embedding, item_embedding):
    uid = x[:, 0]
    iid = x[:, 1]
    return _gcmc_sc_kernel(uid, iid, user_embedding.T, item_embedding.T)


# speedup vs baseline: 1.1502x; 1.0167x over previous
import functools

import jax
import jax.numpy as jnp
from jax import lax
from jax.experimental import pallas as pl
from jax.experimental.pallas import tpu as pltpu
from jax.experimental.pallas import tpu_sc as plsc

B = 16384
D = 32
N_CORES = 2
N_SUBCORES = 16
NW = N_CORES * N_SUBCORES
BPW = B // NW
LANES = 16
NBUF = 8

_mesh = plsc.VectorSubcoreMesh(core_axis_name="c", subcore_axis_name="s")


@functools.partial(
    pl.kernel,
    mesh=_mesh,
    out_type=jax.ShapeDtypeStruct((B,), jnp.float32),
    compiler_params=pltpu.CompilerParams(
        needs_layout_passes=False, use_tc_tiling_on_sc=True),
    scratch_types=[
        pltpu.VMEM((BPW + LANES,), jnp.int32),
        pltpu.VMEM((BPW + LANES,), jnp.int32),
        pltpu.VMEM((NBUF, D, 128), jnp.float32),
        pltpu.VMEM((NBUF, D, 128), jnp.float32),
        pltpu.VMEM((BPW,), jnp.float32),
        pltpu.SemaphoreType.DMA,
        pltpu.SemaphoreType.DMA,
    ],
)
def _gcmc_sc_kernel(uid_hbm, iid_hbm, utT_hbm, itT_hbm, out_hbm,
                    uidx, iidx, ublk, iblk, out_v, usem, isem):
    wid = lax.axis_index("s") * N_CORES + lax.axis_index("c")
    base = wid * BPW

    pltpu.sync_copy(uid_hbm.at[pl.ds(base, BPW)], uidx.at[pl.ds(0, BPW)])
    pltpu.sync_copy(iid_hbm.at[pl.ds(base, BPW)], iidx.at[pl.ds(0, BPW)])

    lane = jnp.arange(LANES, dtype=jnp.int32)

    def issue(uvi, ivi, slot):
        tc_u = pl.multiple_of((uvi >> 7) * 128, 128)
        tc_i = pl.multiple_of((ivi >> 7) * 128, 128)
        pltpu.make_async_copy(
            utT_hbm.at[:, pl.ds(tc_u, 128)], ublk.at[slot], usem).start()
        pltpu.make_async_copy(
            itT_hbm.at[:, pl.ds(tc_i, 128)], iblk.at[slot], isem).start()

    def wait(slot):
        pltpu.make_async_copy(
            utT_hbm.at[:, pl.ds(0, 128)], ublk.at[slot], usem).wait()
        pltpu.make_async_copy(
            itT_hbm.at[:, pl.ds(0, 128)], iblk.at[slot], isem).wait()

    uvec0 = uidx[pl.ds(0, LANES)]
    ivec0 = iidx[pl.ds(0, LANES)]
    for n in range(NBUF):
        issue(uvec0[n], ivec0[n], n)

    def body(g, carry):
        uvec = uidx[pl.ds(g * LANES, LANES)]
        ivec = iidx[pl.ds(g * LANES, LANES)]
        uvec_n = uidx[pl.ds((g + 1) * LANES, LANES)]
        ivec_n = iidx[pl.ds((g + 1) * LANES, LANES)]
        acc = jnp.zeros((LANES,), jnp.float32)
        for i in range(LANES):
            slot = i % NBUF
            lu = jnp.full((LANES,), uvec[i] & 127, jnp.int32)
            li = jnp.full((LANES,), ivec[i] & 127, jnp.int32)

            wait(slot)
            u0 = plsc.load_gather(ublk.at[slot], [lane, lu])
            u1 = plsc.load_gather(ublk.at[slot], [lane + LANES, lu])
            v0 = plsc.load_gather(iblk.at[slot], [lane, li])
            v1 = plsc.load_gather(iblk.at[slot], [lane + LANES, li])
            s = jnp.sum(u0 * v0 + u1 * v1)
            acc = jnp.where(lane == i, s, acc)

            if i + NBUF < LANES:
                issue(uvec[i + NBUF], ivec[i + NBUF], slot)
            else:
                @pl.when(g < BPW // LANES - 1)
                def _():
                    issue(uvec_n[i + NBUF - LANES],
                          ivec_n[i + NBUF - LANES], slot)

        out_v[pl.ds(g * LANES, LANES)] = acc
        return carry

    lax.fori_loop(0, BPW // LANES, body, 0)

    pltpu.sync_copy(out_v, out_hbm.at[pl.ds(base, BPW)])


def kernel(x, user_embedding, item_embedding):
    uid = x[:, 0]
    iid = x[:, 1]
    return _gcmc_sc_kernel(uid, iid, user_embedding.T, item_embedding.T)
